# baseline (speedup 1.0000x reference)
import jax
import jax.numpy as jnp
from jax.experimental import pallas as pl
from jax.experimental.pallas import tpu as pltpu

_VMEM_LIMIT = 48 * 1024 * 1024

_F32 = jnp.float32


def _spec(shp):
    return pl.BlockSpec(shp, lambda: tuple(0 for _ in shp))


def _kh_patch(buf_ref, patch_ref, M, C):
    for kh in range(5):
        patch_ref[:, kh * C:(kh + 1) * C] = (
            buf_ref[:, kh:kh + 16, :, :].reshape(M, C))


def _stage12_kernel(xa_ref, w1a_ref, b1a_ref, w1b_ref, b1b_ref,
                    w2a_ref, b2a_ref, w2b_ref, b2b_ref, o_ref,
                    buf1_ref, patch1_ref,
                    buf2a_ref, patch2a_ref,
                    buf2b_ref, patch2b_ref):
    ya = jnp.dot(xa_ref[...], w1a_ref[...],
                 preferred_element_type=_F32) + b1a_ref[...]
    ya = jnp.maximum(ya, 0.0).astype(jnp.bfloat16)

    buf1_ref[...] = jnp.zeros_like(buf1_ref)
    buf1_ref[:, 2:18, 2:66, :] = ya.reshape(8, 16, 64, 64)
    _kh_patch(buf1_ref, patch1_ref, 8 * 16 * 72, 64)
    z1 = jnp.dot(patch1_ref[...], w1b_ref[...],
                 preferred_element_type=_F32
                 ).reshape(128, 72, 320)
    y = (z1[:, 0:64, 0:64] + z1[:, 1:65, 64:128]
         + z1[:, 2:66, 128:192] + z1[:, 3:67, 192:256]
         + z1[:, 4:68, 256:320])
    y = jnp.maximum(y + b1b_ref[...], 0.0)
    y = jnp.max(y.reshape(128, 16, 4, 64), axis=2)
    y = y.astype(jnp.bfloat16)

    buf2a_ref[...] = jnp.zeros_like(buf2a_ref)
    buf2a_ref[:, 2:18, 2:18, :] = y.reshape(8, 16, 16, 64)
    _kh_patch(buf2a_ref, patch2a_ref, 8 * 16 * 24, 64)
    z2a = jnp.dot(patch2a_ref[...], w2a_ref[...],
                  preferred_element_type=_F32
                  ).reshape(128, 24, 640)
    y = (z2a[:, 0:16, 0:128] + z2a[:, 1:17, 128:256]
         + z2a[:, 2:18, 256:384] + z2a[:, 3:19, 384:512]
         + z2a[:, 4:20, 512:640])
    y = jnp.maximum(y + b2a_ref[...], 0.0).astype(jnp.bfloat16)

    buf2b_ref[...] = jnp.zeros_like(buf2b_ref)
    buf2b_ref[:, 2:18, 2:18, :] = y.reshape(8, 16, 16, 128)
    _kh_patch(buf2b_ref, patch2b_ref, 8 * 16 * 24, 128)
    z2b = jnp.dot(patch2b_ref[...], w2b_ref[...],
                  preferred_element_type=_F32
                  ).reshape(128, 24, 640)
    y = (z2b[:, 0:16, 0:128] + z2b[:, 1:17, 128:256]
         + z2b[:, 2:18, 256:384] + z2b[:, 3:19, 384:512]
         + z2b[:, 4:20, 512:640])
    y = jnp.maximum(y + b2b_ref[...], 0.0)
    y = jnp.max(y.reshape(128, 4, 4, 128), axis=2)
    o_ref[...] = y.reshape(8, 16, 4, 128).astype(o_ref.dtype)


def _stage12(xa, w1a, b1a, w1b, b1b, w2a, b2a, w2b, b2b):
    return pl.pallas_call(
        _stage12_kernel,
        out_shape=jax.ShapeDtypeStruct((8, 16, 4, 128), jnp.bfloat16),
        in_specs=[
            _spec((8192, 32)),
            _spec((32, 64)), _spec((1, 64)),
            _spec((320, 320)), _spec((1, 64)),
            _spec((320, 640)), _spec((1, 128)),
            _spec((640, 640)), _spec((1, 128)),
        ],
        out_specs=_spec((8, 16, 4, 128)),
        scratch_shapes=[
            pltpu.VMEM((8, 20, 72, 64), jnp.bfloat16),
            pltpu.VMEM((8 * 16 * 72, 320), jnp.bfloat16),
            pltpu.VMEM((8, 20, 24, 64), jnp.bfloat16),
            pltpu.VMEM((8 * 16 * 24, 320), jnp.bfloat16),
            pltpu.VMEM((8, 20, 24, 128), jnp.bfloat16),
            pltpu.VMEM((8 * 16 * 24, 640), jnp.bfloat16),
        ],
        compiler_params=pltpu.CompilerParams(
            vmem_limit_bytes=_VMEM_LIMIT,
        ),
    )(xa, w1a, b1a.reshape(1, 64), w1b, b1b.reshape(1, 64),
      w2a, b2a.reshape(1, 128), w2b, b2b.reshape(1, 128))


def _stage3_head_kernel(h2_ref, w3a_ref, b3a_ref, w3b_ref, b3b_ref,
                        w4_ref, b4_ref, wl4_ref, bl4_ref,
                        wl2_ref, bl2_ref, wl3_ref, bl3_ref, o_ref,
                        buf3a_ref, patch3a_ref,
                        buf3b_ref, patch3b_ref,
                        buf4_ref, patch4_ref):
    buf3a_ref[...] = jnp.zeros_like(buf3a_ref)
    buf3a_ref[:, 2:18, 2:6, :] = h2_ref[...]
    _kh_patch(buf3a_ref, patch3a_ref, 8 * 16 * 8, 128)
    z3a = jnp.dot(patch3a_ref[...], w3a_ref[...],
                  preferred_element_type=_F32
                  ).reshape(128, 8, 1280)
    y = (z3a[:, 0:4, 0:256] + z3a[:, 1:5, 256:512]
         + z3a[:, 2:6, 512:768] + z3a[:, 3:7, 768:1024]
         + z3a[:, 4:8, 1024:1280])
    y = jnp.maximum(y + b3a_ref[...], 0.0).astype(jnp.bfloat16)

    buf3b_ref[...] = jnp.zeros_like(buf3b_ref)
    buf3b_ref[:, 2:18, 2:6, :] = y.reshape(8, 16, 4, 256)
    _kh_patch(buf3b_ref, patch3b_ref, 8 * 16 * 8, 256)
    z3b = jnp.dot(patch3b_ref[...], w3b_ref[...],
                  preferred_element_type=_F32
                  ).reshape(128, 8, 1280)
    y = (z3b[:, 0:4, 0:256] + z3b[:, 1:5, 256:512]
         + z3b[:, 2:6, 512:768] + z3b[:, 3:7, 768:1024]
         + z3b[:, 4:8, 1024:1280])
    y = jnp.maximum(y + b3b_ref[...], 0.0)
    y = jnp.max(y, axis=1).astype(jnp.bfloat16)

    buf4_ref[...] = jnp.zeros_like(buf4_ref)
    buf4_ref[:, 2:18, :] = y.reshape(8, 16, 256)
    for kh in range(5):
        patch4_ref[:, kh * 256:(kh + 1) * 256] = (
            buf4_ref[:, kh:kh + 16, :].reshape(128, 256))
    f = jnp.dot(patch4_ref[...], w4_ref[...],
                preferred_element_type=_F32) + b4_ref[...]
    f = jnp.max(f.reshape(8, 16, 2048), axis=1).astype(jnp.bfloat16)

    h = jnp.dot(f, wl4_ref[...],
                preferred_element_type=_F32) + bl4_ref[...]
    h = jnp.maximum(h, 0.0).astype(jnp.bfloat16)
    h = jnp.dot(h, wl2_ref[...],
                preferred_element_type=_F32) + bl2_ref[...]
    h = jnp.maximum(h, 0.0).astype(jnp.bfloat16)
    z = jnp.dot(h, wl3_ref[...],
                preferred_element_type=_F32) + bl3_ref[...]
    z = z - jnp.max(z, axis=-1, keepdims=True)
    o_ref[...] = z - jnp.log(jnp.sum(jnp.exp(z), axis=-1, keepdims=True))


def _stage3_head(h2, w3a, b3a, w3b, b3b, w4s, b4, wl4, bl4, wl2, bl2, wl3, bl3):
    return pl.pallas_call(
        _stage3_head_kernel,
        out_shape=jax.ShapeDtypeStruct((8, 16), jnp.float32),
        in_specs=[
            _spec((8, 16, 4, 128)),
            _spec((640, 1280)), _spec((1, 256)),
            _spec((1280, 1280)), _spec((1, 256)),
            _spec((1280, 2048)), _spec((1, 2048)),
            _spec((2048, 512)), _spec((1, 512)),
            _spec((512, 1024)), _spec((1, 1024)),
            _spec((1024, 16)), _spec((1, 16)),
        ],
        out_specs=_spec((8, 16)),
        scratch_shapes=[
            pltpu.VMEM((8, 20, 8, 128), jnp.bfloat16),
            pltpu.VMEM((8 * 16 * 8, 640), jnp.bfloat16),
            pltpu.VMEM((8, 20, 8, 256), jnp.bfloat16),
            pltpu.VMEM((8 * 16 * 8, 1280), jnp.bfloat16),
            pltpu.VMEM((8, 24, 256), jnp.bfloat16),
            pltpu.VMEM((128, 1280), jnp.bfloat16),
        ],
        compiler_params=pltpu.CompilerParams(
            vmem_limit_bytes=_VMEM_LIMIT,
        ),
    )(h2, w3a, b3a.reshape(1, 256), w3b, b3b.reshape(1, 256),
      w4s, b4.reshape(1, 2048), wl4, bl4.reshape(1, 512),
      wl2, bl2.reshape(1, 1024), wl3, bl3.reshape(1, 16))


def _fused_kernel(xa_ref, w1a_ref, b1a_ref, w1b_ref, b1b_ref,
                  w2a_ref, b2a_ref, w2b_ref, b2b_ref,
                  w3a_ref, b3a_ref, w3b_ref, b3b_ref,
                  w4_ref, b4_ref, wl4_ref, bl4_ref,
                  wl2_ref, bl2_ref, wl3_ref, bl3_ref, o_ref,
                  buf1_ref, patch1_ref, buf2a_ref, patch2a_ref,
                  buf2b_ref, patch2b_ref, h2_ref, buf3a_ref, patch3a_ref,
                  buf3b_ref, patch3b_ref, buf4_ref, patch4_ref):
    _stage12_kernel(xa_ref, w1a_ref, b1a_ref, w1b_ref, b1b_ref,
                    w2a_ref, b2a_ref, w2b_ref, b2b_ref, h2_ref,
                    buf1_ref, patch1_ref, buf2a_ref, patch2a_ref,
                    buf2b_ref, patch2b_ref)
    _stage3_head_kernel(h2_ref, w3a_ref, b3a_ref, w3b_ref, b3b_ref,
                        w4_ref, b4_ref, wl4_ref, bl4_ref,
                        wl2_ref, bl2_ref, wl3_ref, bl3_ref, o_ref,
                        buf3a_ref, patch3a_ref,
                        buf3b_ref, patch3b_ref,
                        buf4_ref, patch4_ref)


def _fused(xa, w1a, b1a, w1b, b1b, w2a, b2a, w2b, b2b,
           w3a, b3a, w3b, b3b, w4s, b4, wl4, bl4, wl2, bl2, wl3, bl3):
    return pl.pallas_call(
        _fused_kernel,
        out_shape=jax.ShapeDtypeStruct((8, 16), jnp.float32),
        in_specs=[
            _spec((8192, 32)),
            _spec((32, 64)), _spec((1, 64)),
            _spec((320, 320)), _spec((1, 64)),
            _spec((320, 640)), _spec((1, 128)),
            _spec((640, 640)), _spec((1, 128)),
            _spec((640, 1280)), _spec((1, 256)),
            _spec((1280, 1280)), _spec((1, 256)),
            _spec((1280, 2048)), _spec((1, 2048)),
            _spec((2048, 512)), _spec((1, 512)),
            _spec((512, 1024)), _spec((1, 1024)),
            _spec((1024, 16)), _spec((1, 16)),
        ],
        out_specs=_spec((8, 16)),
        scratch_shapes=[
            pltpu.VMEM((8, 20, 72, 64), jnp.bfloat16),
            pltpu.VMEM((8 * 16 * 72, 320), jnp.bfloat16),
            pltpu.VMEM((8, 20, 24, 64), jnp.bfloat16),
            pltpu.VMEM((8 * 16 * 24, 320), jnp.bfloat16),
            pltpu.VMEM((8, 20, 24, 128), jnp.bfloat16),
            pltpu.VMEM((8 * 16 * 24, 640), jnp.bfloat16),
            pltpu.VMEM((8, 16, 4, 128), jnp.bfloat16),
            pltpu.VMEM((8, 20, 8, 128), jnp.bfloat16),
            pltpu.VMEM((8 * 16 * 8, 640), jnp.bfloat16),
            pltpu.VMEM((8, 20, 8, 256), jnp.bfloat16),
            pltpu.VMEM((8 * 16 * 8, 1280), jnp.bfloat16),
            pltpu.VMEM((8, 24, 256), jnp.bfloat16),
            pltpu.VMEM((128, 1280), jnp.bfloat16),
        ],
        compiler_params=pltpu.CompilerParams(
            vmem_limit_bytes=56 * 1024 * 1024,
        ),
    )(xa, w1a, b1a.reshape(1, 64), w1b, b1b.reshape(1, 64),
      w2a, b2a.reshape(1, 128), w2b, b2b.reshape(1, 128),
      w3a, b3a.reshape(1, 256), w3b, b3b.reshape(1, 256),
      w4s, b4.reshape(1, 2048), wl4, bl4.reshape(1, 512),
      wl2, bl2.reshape(1, 1024), wl3, bl3.reshape(1, 16))


def _taps_1ch(x):
    B, H, W = x.shape
    xp = jnp.pad(x, ((0, 0), (2, 2), (2, 2)))
    cols = [xp[:, kh:kh + H, kw:kw + W] for kh in range(5) for kw in range(5)]
    taps = jnp.stack(cols, axis=-1)
    taps = jnp.pad(taps, ((0, 0), (0, 0), (0, 0), (0, 7)))
    return taps.astype(jnp.bfloat16).reshape(B * H * W, 32)


def _cat_kw(w, cin, cout):
    return w.reshape(5, 5, cin, cout).transpose(0, 2, 1, 3).reshape(
        5 * cin, 5 * cout)


def kernel(x, conv1a_w, conv1a_b, conv1b_w, conv1b_b, conv2a_w, conv2a_b,
           conv2b_w, conv2b_b, conv3a_w, conv3a_b, conv3b_w, conv3b_b,
           conv4_w, conv4_b, line4_w, line4_b, line2_w, line2_b,
           line3_w, line3_b):
    xa = _taps_1ch(x)
    w4s = conv4_w.reshape(5, 5, 256, 2048)[:, 2].reshape(1280, 2048)
    return _fused(xa, conv1a_w, conv1a_b,
                  _cat_kw(conv1b_w, 64, 64), conv1b_b,
                  _cat_kw(conv2a_w, 64, 128), conv2a_b,
                  _cat_kw(conv2b_w, 128, 128), conv2b_b,
                  _cat_kw(conv3a_w, 128, 256), conv3a_b,
                  _cat_kw(conv3b_w, 256, 256), conv3b_b,
                  w4s, conv4_b,
                  line4_w, line4_b, line2_w, line2_b, line3_w, line3_b)


# speedup vs baseline: 1.1356x; 1.1356x over previous
import jax
import jax.numpy as jnp
from jax.experimental import pallas as pl
from jax.experimental.pallas import tpu as pltpu

_VMEM_LIMIT = 48 * 1024 * 1024

_F32 = jnp.float32


def _spec(shp):
    return pl.BlockSpec(shp, lambda: tuple(0 for _ in shp))


def _kh_patch(buf_ref, patch_ref, M, C):
    for kh in range(5):
        patch_ref[:, kh * C:(kh + 1) * C] = (
            buf_ref[:, kh:kh + 16, :, :].reshape(M, C))


def _stage12_kernel(xa_ref, w1a_ref, b1a_ref, w1b_ref, b1b_ref,
                    w2a_ref, b2a_ref, w2b_ref, b2b_ref, o_ref,
                    buf1_ref, patch1_ref,
                    buf2a_ref, patch2a_ref,
                    buf2b_ref, patch2b_ref):
    ya = jnp.dot(xa_ref[...], w1a_ref[...],
                 preferred_element_type=_F32) + b1a_ref[...]
    ya = jnp.maximum(ya, 0.0).astype(jnp.bfloat16)

    buf1_ref[...] = jnp.zeros_like(buf1_ref)
    buf1_ref[:, 2:18, 2:66, :] = ya.reshape(8, 16, 64, 64)
    _kh_patch(buf1_ref, patch1_ref, 8 * 16 * 72, 64)
    z1 = jnp.dot(patch1_ref[...], w1b_ref[...],
                 preferred_element_type=_F32
                 ).reshape(128, 72, 320)
    y = (z1[:, 0:64, 0:64] + z1[:, 1:65, 64:128]
         + z1[:, 2:66, 128:192] + z1[:, 3:67, 192:256]
         + z1[:, 4:68, 256:320])
    y = jnp.maximum(y + b1b_ref[...], 0.0)
    y = jnp.max(y.reshape(128, 16, 4, 64), axis=2)
    y = y.astype(jnp.bfloat16)

    buf2a_ref[...] = jnp.zeros_like(buf2a_ref)
    buf2a_ref[:, 2:18, 2:18, :] = y.reshape(8, 16, 16, 64)
    _kh_patch(buf2a_ref, patch2a_ref, 8 * 16 * 24, 64)
    z2a = jnp.dot(patch2a_ref[...], w2a_ref[...],
                  preferred_element_type=_F32
                  ).reshape(128, 24, 640)
    y = (z2a[:, 0:16, 0:128] + z2a[:, 1:17, 128:256]
         + z2a[:, 2:18, 256:384] + z2a[:, 3:19, 384:512]
         + z2a[:, 4:20, 512:640])
    y = jnp.maximum(y + b2a_ref[...], 0.0).astype(jnp.bfloat16)

    buf2b_ref[...] = jnp.zeros_like(buf2b_ref)
    buf2b_ref[:, 2:18, 2:18, :] = y.reshape(8, 16, 16, 128)
    _kh_patch(buf2b_ref, patch2b_ref, 8 * 16 * 24, 128)
    z2b = jnp.dot(patch2b_ref[...], w2b_ref[...],
                  preferred_element_type=_F32
                  ).reshape(128, 24, 640)
    y = (z2b[:, 0:16, 0:128] + z2b[:, 1:17, 128:256]
         + z2b[:, 2:18, 256:384] + z2b[:, 3:19, 384:512]
         + z2b[:, 4:20, 512:640])
    y = jnp.maximum(y + b2b_ref[...], 0.0)
    y = jnp.max(y.reshape(128, 4, 4, 128), axis=2)
    o_ref[...] = y.reshape(8, 16, 4, 128).astype(o_ref.dtype)


def _stage12(xa, w1a, b1a, w1b, b1b, w2a, b2a, w2b, b2b):
    return pl.pallas_call(
        _stage12_kernel,
        out_shape=jax.ShapeDtypeStruct((8, 16, 4, 128), jnp.bfloat16),
        in_specs=[
            _spec((8192, 32)),
            _spec((32, 64)), _spec((1, 64)),
            _spec((320, 320)), _spec((1, 64)),
            _spec((320, 640)), _spec((1, 128)),
            _spec((640, 640)), _spec((1, 128)),
        ],
        out_specs=_spec((8, 16, 4, 128)),
        scratch_shapes=[
            pltpu.VMEM((8, 20, 72, 64), jnp.bfloat16),
            pltpu.VMEM((8 * 16 * 72, 320), jnp.bfloat16),
            pltpu.VMEM((8, 20, 24, 64), jnp.bfloat16),
            pltpu.VMEM((8 * 16 * 24, 320), jnp.bfloat16),
            pltpu.VMEM((8, 20, 24, 128), jnp.bfloat16),
            pltpu.VMEM((8 * 16 * 24, 640), jnp.bfloat16),
        ],
        compiler_params=pltpu.CompilerParams(
            vmem_limit_bytes=_VMEM_LIMIT,
        ),
    )(xa, w1a, b1a.reshape(1, 64), w1b, b1b.reshape(1, 64),
      w2a, b2a.reshape(1, 128), w2b, b2b.reshape(1, 128))


def _stage3_head_kernel(h2_ref, w3a_ref, b3a_ref, w3b_ref, b3b_ref,
                        w4_ref, b4_ref, wl4_ref, bl4_ref,
                        wl2_ref, bl2_ref, wl3_ref, bl3_ref, o_ref,
                        buf3a_ref, patch3a_ref,
                        buf3b_ref, patch3b_ref,
                        buf4_ref, patch4_ref):
    buf3a_ref[...] = jnp.zeros_like(buf3a_ref)
    buf3a_ref[:, 2:18, 2:6, :] = h2_ref[...]
    _kh_patch(buf3a_ref, patch3a_ref, 8 * 16 * 8, 128)
    z3a = jnp.dot(patch3a_ref[...], w3a_ref[...],
                  preferred_element_type=_F32
                  ).reshape(128, 8, 1280)
    y = (z3a[:, 0:4, 0:256] + z3a[:, 1:5, 256:512]
         + z3a[:, 2:6, 512:768] + z3a[:, 3:7, 768:1024]
         + z3a[:, 4:8, 1024:1280])
    y = jnp.maximum(y + b3a_ref[...], 0.0).astype(jnp.bfloat16)

    buf3b_ref[...] = jnp.zeros_like(buf3b_ref)
    buf3b_ref[:, 2:18, 2:6, :] = y.reshape(8, 16, 4, 256)
    _kh_patch(buf3b_ref, patch3b_ref, 8 * 16 * 8, 256)
    z3b = jnp.dot(patch3b_ref[...], w3b_ref[...],
                  preferred_element_type=_F32
                  ).reshape(128, 8, 1280)
    y = (z3b[:, 0:4, 0:256] + z3b[:, 1:5, 256:512]
         + z3b[:, 2:6, 512:768] + z3b[:, 3:7, 768:1024]
         + z3b[:, 4:8, 1024:1280])
    y = jnp.maximum(y + b3b_ref[...], 0.0)
    y = jnp.max(y, axis=1).astype(jnp.bfloat16)

    buf4_ref[...] = jnp.zeros_like(buf4_ref)
    buf4_ref[:, 2:18, :] = y.reshape(8, 16, 256)
    for kh in range(5):
        patch4_ref[:, kh * 256:(kh + 1) * 256] = (
            buf4_ref[:, kh:kh + 16, :].reshape(128, 256))
    f = jnp.dot(patch4_ref[...], w4_ref[...],
                preferred_element_type=_F32) + b4_ref[...]
    f = jnp.max(f.reshape(8, 16, 2048), axis=1).astype(jnp.bfloat16)

    h = jnp.dot(f, wl4_ref[...],
                preferred_element_type=_F32) + bl4_ref[...]
    h = jnp.maximum(h, 0.0).astype(jnp.bfloat16)
    h = jnp.dot(h, wl2_ref[...],
                preferred_element_type=_F32) + bl2_ref[...]
    h = jnp.maximum(h, 0.0).astype(jnp.bfloat16)
    z = jnp.dot(h, wl3_ref[...],
                preferred_element_type=_F32) + bl3_ref[...]
    z = z - jnp.max(z, axis=-1, keepdims=True)
    o_ref[...] = z - jnp.log(jnp.sum(jnp.exp(z), axis=-1, keepdims=True))


def _stage3_head(h2, w3a, b3a, w3b, b3b, w4s, b4, wl4, bl4, wl2, bl2, wl3, bl3):
    return pl.pallas_call(
        _stage3_head_kernel,
        out_shape=jax.ShapeDtypeStruct((8, 16), jnp.float32),
        in_specs=[
            _spec((8, 16, 4, 128)),
            _spec((640, 1280)), _spec((1, 256)),
            _spec((1280, 1280)), _spec((1, 256)),
            _spec((1280, 2048)), _spec((1, 2048)),
            _spec((2048, 512)), _spec((1, 512)),
            _spec((512, 1024)), _spec((1, 1024)),
            _spec((1024, 16)), _spec((1, 16)),
        ],
        out_specs=_spec((8, 16)),
        scratch_shapes=[
            pltpu.VMEM((8, 20, 8, 128), jnp.bfloat16),
            pltpu.VMEM((8 * 16 * 8, 640), jnp.bfloat16),
            pltpu.VMEM((8, 20, 8, 256), jnp.bfloat16),
            pltpu.VMEM((8 * 16 * 8, 1280), jnp.bfloat16),
            pltpu.VMEM((8, 24, 256), jnp.bfloat16),
            pltpu.VMEM((128, 1280), jnp.bfloat16),
        ],
        compiler_params=pltpu.CompilerParams(
            vmem_limit_bytes=_VMEM_LIMIT,
        ),
    )(h2, w3a, b3a.reshape(1, 256), w3b, b3b.reshape(1, 256),
      w4s, b4.reshape(1, 2048), wl4, bl4.reshape(1, 512),
      wl2, bl2.reshape(1, 1024), wl3, bl3.reshape(1, 16))


def _taps_1ch(x):
    B, H, W = x.shape
    xp = jnp.pad(x, ((0, 0), (2, 2), (2, 2)))
    cols = [xp[:, kh:kh + H, kw:kw + W] for kh in range(5) for kw in range(5)]
    taps = jnp.stack(cols, axis=-1)
    taps = jnp.pad(taps, ((0, 0), (0, 0), (0, 0), (0, 7)))
    return taps.astype(jnp.bfloat16).reshape(B * H * W, 32)


def _cat_kw(w, cin, cout):
    return w.reshape(5, 5, cin, cout).transpose(0, 2, 1, 3).reshape(
        5 * cin, 5 * cout)


def kernel(x, conv1a_w, conv1a_b, conv1b_w, conv1b_b, conv2a_w, conv2a_b,
           conv2b_w, conv2b_b, conv3a_w, conv3a_b, conv3b_w, conv3b_b,
           conv4_w, conv4_b, line4_w, line4_b, line2_w, line2_b,
           line3_w, line3_b):
    xa = _taps_1ch(x)
    h2 = _stage12(xa, conv1a_w, conv1a_b,
                  _cat_kw(conv1b_w, 64, 64), conv1b_b,
                  _cat_kw(conv2a_w, 64, 128), conv2a_b,
                  _cat_kw(conv2b_w, 128, 128), conv2b_b)
    w4s = conv4_w.reshape(5, 5, 256, 2048)[:, 2].reshape(1280, 2048)
    return _stage3_head(h2,
                        _cat_kw(conv3a_w, 128, 256), conv3a_b,
                        _cat_kw(conv3b_w, 256, 256), conv3b_b,
                        w4s, conv4_b,
                        line4_w, line4_b, line2_w, line2_b,
                        line3_w, line3_b)
